# baseline (speedup 1.0000x reference)
import functools

import jax
import jax.numpy as jnp
from jax.experimental import pallas as pl
from jax.experimental.pallas import tpu as pltpu


def _mm_body(adj_hbm, w_ref, out_ref, buf, sem, *, bm, nbuf, nblocks):
    def cp(i, slot):
        return pltpu.make_async_copy(
            adj_hbm.at[pl.ds(i * bm, bm), :], buf.at[slot], sem.at[slot]
        )

    for s in range(nbuf - 1):
        cp(s, s).start()

    def step(i, carry):
        nxt = i + (nbuf - 1)

        @pl.when(nxt < nblocks)
        def _():
            cp(nxt, jax.lax.rem(nxt, nbuf)).start()

        slot = jax.lax.rem(i, nbuf)
        cp(i, slot).wait()
        out_ref[pl.ds(i * bm, bm), :] = jnp.dot(
            buf[slot], w_ref[...], preferred_element_type=jnp.float32
        )
        return carry

    jax.lax.fori_loop(0, nblocks, step, 0)


def kernel(adj, weight):
    m, k = adj.shape
    k2, n = weight.shape
    assert k == k2
    bm = 128
    nbuf = 4
    nblocks = m // bm
    return pl.pallas_call(
        functools.partial(_mm_body, bm=bm, nbuf=nbuf, nblocks=nblocks),
        in_specs=[
            pl.BlockSpec(memory_space=pltpu.HBM),
            pl.BlockSpec((k2, n), lambda: (0, 0)),
        ],
        out_specs=pl.BlockSpec((m, n), lambda: (0, 0)),
        out_shape=jax.ShapeDtypeStruct((m, n), jnp.float32),
        scratch_shapes=[
            pltpu.VMEM((nbuf, bm, k), jnp.float32),
            pltpu.SemaphoreType.DMA((nbuf,)),
        ],
    )(adj, weight)
